# baseline (speedup 1.0000x reference)
import functools

import numpy as np

import jax
import jax.numpy as jnp
from jax.experimental import pallas as pl
from jax.experimental.pallas import tpu as pltpu

_QTILE = 128
_NSTRIPE = 32
_SCOLS = _NSTRIPE * 16
_NHEAD = 2
_NBUF = 4
_NEG = -1e30


def _round128(n: int) -> int:
    return -(-n // 128) * 128


def _local_bias() -> np.ndarray:
    j = np.arange(_QTILE)[:, None] // 16
    k = np.arange(_QTILE)[None, :] // 16
    valid = ((j // 4 == k // 4) & (k <= j)) | ((k == 3) & (j >= 4))
    return np.where(valid, 0.0, _NEG).astype(np.float32)


def _stripe_bias(nt: int) -> np.ndarray:
    t = np.arange(nt)[:, None]
    col = np.arange(_SCOLS)[None, :]
    return np.where(col < 32 * t, 0.0, _NEG).astype(np.float32)


def _make_body(ntiles: int):
    def _attn_body(bl_ref, bs_ref, q_hbm, k_hbm, v_hbm, o_ref,
                   qbuf, kbuf, vbuf, ks_ref, vs_ref, sem):
        i = pl.program_id(0)
        ni = pl.num_programs(0)
        slot = jax.lax.rem(i, _NBUF)

        def _startcp(src_i, dst_slot):
            pltpu.make_async_copy(
                q_hbm.at[src_i], qbuf.at[dst_slot], sem.at[dst_slot, 0]
            ).start()
            pltpu.make_async_copy(
                k_hbm.at[src_i], kbuf.at[dst_slot], sem.at[dst_slot, 1]
            ).start()
            pltpu.make_async_copy(
                v_hbm.at[src_i], vbuf.at[dst_slot], sem.at[dst_slot, 2]
            ).start()

        @pl.when(i == 0)
        def _bootstrap():
            _startcp(0, 0)

            @pl.when(ni > 1)
            def _second():
                _startcp(1, 1)

        @pl.when(jnp.logical_and(i == 0, ni > 2))
        def _third():
            _startcp(2, 2)

        @pl.when(i + 3 < ni)
        def _prefetch():
            _startcp(i + 3, jax.lax.rem(i + 3, _NBUF))

        pltpu.make_async_copy(
            q_hbm.at[i], qbuf.at[slot], sem.at[slot, 0]).wait()
        pltpu.make_async_copy(
            k_hbm.at[i], kbuf.at[slot], sem.at[slot, 1]).wait()
        pltpu.make_async_copy(
            v_hbm.at[i], vbuf.at[slot], sem.at[slot, 2]).wait()

        q_ref = qbuf.at[slot]
        k_ref = kbuf.at[slot]
        v_ref = vbuf.at[slot]

        for hh in range(_NHEAD):
            for kk in range(_NSTRIPE):
                src = kk * 64 + 48
                dst = kk * 16
                ks_ref[hh, dst:dst + 16, :] = k_ref[hh, src:src + 16, :]
                vs_ref[hh, dst:dst + 16, :] = (
                    v_ref[hh, src:src + 16, :].astype(jnp.bfloat16))

        scale = q_hbm.shape[-1] ** -0.5
        bias_loc = bl_ref[...]

        def _qk(v):
            hh, t = v % _NHEAD, v // _NHEAD
            lo = t * _QTILE
            q = q_ref[hh, lo:lo + _QTILE, :] * scale
            s_loc = jax.lax.dot_general(
                q, k_ref[hh, lo:lo + _QTILE, :], (((1,), (1,)), ((), ())),
                preferred_element_type=jnp.float32) + bias_loc
            w = _round128(32 * t)
            s_str = None
            if w:
                s_str = jax.lax.dot_general(
                    q, ks_ref[hh, 0:w, :], (((1,), (1,)), ((), ())),
                    preferred_element_type=jnp.float32) + bs_ref[t:t + 1, 0:w]
            return s_loc, s_str, w

        def _sv(v, scores):
            s_loc, s_str, w = scores
            hh, t = v % _NHEAD, v // _NHEAD
            lo = t * _QTILE
            m = jnp.max(s_loc, axis=1, keepdims=True)
            if w:
                m = jnp.maximum(m, jnp.max(s_str, axis=1, keepdims=True))
            e_loc = jnp.exp(s_loc - m)
            denom = jnp.sum(e_loc, axis=1, keepdims=True)
            out = jax.lax.dot_general(
                e_loc.astype(jnp.bfloat16),
                v_ref[hh, lo:lo + _QTILE, :].astype(jnp.bfloat16),
                (((1,), (0,)), ((), ())),
                preferred_element_type=jnp.float32)
            if w:
                e_str = jnp.exp(s_str - m)
                denom += jnp.sum(e_str, axis=1, keepdims=True)
                out += jax.lax.dot_general(
                    e_str.astype(jnp.bfloat16), vs_ref[hh, 0:w, :],
                    (((1,), (0,)), ((), ())),
                    preferred_element_type=jnp.float32)
            o_ref[0, hh, lo:lo + _QTILE, :] = out * (1.0 / denom)

        nv = _NHEAD * ntiles
        depth = 4
        pipe = [_qk(x) for x in range(depth)]
        for v in range(nv):
            nxt = _qk(v + depth) if v + depth < nv else None
            _sv(v, pipe[0])
            pipe = pipe[1:] + [nxt]

    return _attn_body


@functools.partial(jax.jit, static_argnames=())
def kernel(query, key, value, mask):
    del mask
    b, h, s, dh = query.shape
    bh = b * h
    g = bh // _NHEAD
    ntiles = s // _QTILE
    q3 = query.reshape(g, _NHEAD, s, dh)
    k3 = key.reshape(g, _NHEAD, s, dh)
    v3 = value.reshape(g, _NHEAD, s, dh)
    bias_loc = jnp.asarray(_local_bias())
    bias_str = jnp.asarray(_stripe_bias(ntiles))

    out = pl.pallas_call(
        _make_body(ntiles),
        grid=(g,),
        in_specs=[
            pl.BlockSpec((_QTILE, _QTILE), lambda i: (0, 0)),
            pl.BlockSpec((ntiles, _SCOLS), lambda i: (0, 0)),
            pl.BlockSpec(memory_space=pltpu.MemorySpace.HBM),
            pl.BlockSpec(memory_space=pltpu.MemorySpace.HBM),
            pl.BlockSpec(memory_space=pltpu.MemorySpace.HBM),
        ],
        out_specs=pl.BlockSpec((1, _NHEAD, s, dh), lambda i: (i, 0, 0, 0)),
        out_shape=jax.ShapeDtypeStruct((g, _NHEAD, s, dh), jnp.float32),
        scratch_shapes=[
            pltpu.VMEM((_NBUF, _NHEAD, s, dh), jnp.float32),
            pltpu.VMEM((_NBUF, _NHEAD, s, dh), jnp.float32),
            pltpu.VMEM((_NBUF, _NHEAD, s, dh), jnp.float32),
            pltpu.VMEM((_NHEAD, _SCOLS, dh), jnp.float32),
            pltpu.VMEM((_NHEAD, _SCOLS, dh), jnp.bfloat16),
            pltpu.SemaphoreType.DMA((_NBUF, 3)),
        ],
        compiler_params=pltpu.CompilerParams(
            dimension_semantics=("arbitrary",)),
    )(bias_loc, bias_str, q3, k3, v3)
    return out.reshape(b, h, s, dh)


# speedup vs baseline: 1.0005x; 1.0005x over previous
import functools

import numpy as np

import jax
import jax.numpy as jnp
from jax.experimental import pallas as pl
from jax.experimental.pallas import tpu as pltpu

_QTILE = 128
_NSTRIPE = 32
_SCOLS = _NSTRIPE * 16
_NHEAD = 2
_NBUF = 3
_NEG = -1e30


def _round128(n: int) -> int:
    return -(-n // 128) * 128


def _local_bias() -> np.ndarray:
    j = np.arange(_QTILE)[:, None] // 16
    k = np.arange(_QTILE)[None, :] // 16
    valid = ((j // 4 == k // 4) & (k <= j)) | ((k == 3) & (j >= 4))
    return np.where(valid, 0.0, _NEG).astype(np.float32)


def _stripe_bias(nt: int) -> np.ndarray:
    t = np.arange(nt)[:, None]
    col = np.arange(_SCOLS)[None, :]
    return np.where(col < 32 * t, 0.0, _NEG).astype(np.float32)


def _make_body(ntiles: int):
    def _attn_body(bl_ref, bs_ref, q_hbm, k_hbm, v_hbm, o_ref,
                   qbuf, kbuf, vbuf, ks_ref, vs_ref, sem):
        i = pl.program_id(0)
        ni = pl.num_programs(0)
        slot = jax.lax.rem(i, _NBUF)

        def _startcp(src_i, dst_slot):
            pltpu.make_async_copy(
                q_hbm.at[src_i], qbuf.at[dst_slot], sem.at[dst_slot, 0]
            ).start()
            pltpu.make_async_copy(
                k_hbm.at[src_i], kbuf.at[dst_slot], sem.at[dst_slot, 1]
            ).start()
            pltpu.make_async_copy(
                v_hbm.at[src_i], vbuf.at[dst_slot], sem.at[dst_slot, 2]
            ).start()

        @pl.when(i == 0)
        def _bootstrap():
            _startcp(0, 0)

            @pl.when(ni > 1)
            def _second():
                _startcp(1, 1)

        @pl.when(i + 2 < ni)
        def _prefetch():
            _startcp(i + 2, jax.lax.rem(i + 2, _NBUF))

        pltpu.make_async_copy(
            q_hbm.at[i], qbuf.at[slot], sem.at[slot, 0]).wait()
        pltpu.make_async_copy(
            k_hbm.at[i], kbuf.at[slot], sem.at[slot, 1]).wait()
        pltpu.make_async_copy(
            v_hbm.at[i], vbuf.at[slot], sem.at[slot, 2]).wait()

        q_ref = qbuf.at[slot]
        k_ref = kbuf.at[slot]
        v_ref = vbuf.at[slot]

        for hh in range(_NHEAD):
            for kk in range(_NSTRIPE):
                src = kk * 64 + 48
                dst = kk * 16
                ks_ref[hh, dst:dst + 16, :] = k_ref[hh, src:src + 16, :]
                vs_ref[hh, dst:dst + 16, :] = (
                    v_ref[hh, src:src + 16, :].astype(jnp.bfloat16))

        scale = q_hbm.shape[-1] ** -0.5
        bias_loc = bl_ref[...]

        def _qk(v):
            hh, t = v % _NHEAD, v // _NHEAD
            lo = t * _QTILE
            q = q_ref[hh, lo:lo + _QTILE, :] * scale
            s_loc = jax.lax.dot_general(
                q, k_ref[hh, lo:lo + _QTILE, :], (((1,), (1,)), ((), ())),
                preferred_element_type=jnp.float32) + bias_loc
            w = _round128(32 * t)
            s_str = None
            if w:
                s_str = jax.lax.dot_general(
                    q, ks_ref[hh, 0:w, :], (((1,), (1,)), ((), ())),
                    preferred_element_type=jnp.float32) + bs_ref[t:t + 1, 0:w]
            return s_loc, s_str, w

        def _sv(v, scores):
            s_loc, s_str, w = scores
            hh, t = v % _NHEAD, v // _NHEAD
            lo = t * _QTILE
            m = jnp.max(s_loc, axis=1, keepdims=True)
            if w:
                m = jnp.maximum(m, jnp.max(s_str, axis=1, keepdims=True))
            e_loc = jnp.exp(s_loc - m)
            denom = jnp.sum(e_loc, axis=1, keepdims=True)
            out = jax.lax.dot_general(
                e_loc.astype(jnp.bfloat16),
                v_ref[hh, lo:lo + _QTILE, :].astype(jnp.bfloat16),
                (((1,), (0,)), ((), ())),
                preferred_element_type=jnp.float32)
            if w:
                e_str = jnp.exp(s_str - m)
                denom += jnp.sum(e_str, axis=1, keepdims=True)
                out += jax.lax.dot_general(
                    e_str.astype(jnp.bfloat16), vs_ref[hh, 0:w, :],
                    (((1,), (0,)), ((), ())),
                    preferred_element_type=jnp.float32)
            o_ref[0, hh, lo:lo + _QTILE, :] = out * (1.0 / denom)

        nv = _NHEAD * ntiles
        depth = 4
        pipe = [_qk(x) for x in range(depth)]
        for v in range(nv):
            nxt = _qk(v + depth) if v + depth < nv else None
            _sv(v, pipe[0])
            pipe = pipe[1:] + [nxt]

    return _attn_body


@functools.partial(jax.jit, static_argnames=())
def kernel(query, key, value, mask):
    del mask
    b, h, s, dh = query.shape
    bh = b * h
    g = bh // _NHEAD
    ntiles = s // _QTILE
    q3 = query.reshape(g, _NHEAD, s, dh)
    k3 = key.reshape(g, _NHEAD, s, dh)
    v3 = value.reshape(g, _NHEAD, s, dh)
    bias_loc = jnp.asarray(_local_bias())
    bias_str = jnp.asarray(_stripe_bias(ntiles))

    out = pl.pallas_call(
        _make_body(ntiles),
        grid=(g,),
        in_specs=[
            pl.BlockSpec((_QTILE, _QTILE), lambda i: (0, 0)),
            pl.BlockSpec((ntiles, _SCOLS), lambda i: (0, 0)),
            pl.BlockSpec(memory_space=pltpu.MemorySpace.HBM),
            pl.BlockSpec(memory_space=pltpu.MemorySpace.HBM),
            pl.BlockSpec(memory_space=pltpu.MemorySpace.HBM),
        ],
        out_specs=pl.BlockSpec((1, _NHEAD, s, dh), lambda i: (i, 0, 0, 0)),
        out_shape=jax.ShapeDtypeStruct((g, _NHEAD, s, dh), jnp.float32),
        scratch_shapes=[
            pltpu.VMEM((_NBUF, _NHEAD, s, dh), jnp.float32),
            pltpu.VMEM((_NBUF, _NHEAD, s, dh), jnp.float32),
            pltpu.VMEM((_NBUF, _NHEAD, s, dh), jnp.float32),
            pltpu.VMEM((_NHEAD, _SCOLS, dh), jnp.float32),
            pltpu.VMEM((_NHEAD, _SCOLS, dh), jnp.bfloat16),
            pltpu.SemaphoreType.DMA((_NBUF, 3)),
        ],
        compiler_params=pltpu.CompilerParams(
            dimension_semantics=("arbitrary",)),
    )(bias_loc, bias_str, q3, k3, v3)
    return out.reshape(b, h, s, dh)
